# baseline (speedup 1.0000x reference)
import functools

import jax
import jax.numpy as jnp
from jax import lax
from jax.experimental import pallas as pl
from jax.experimental.pallas import tpu as pltpu
from jax.experimental.pallas import tpu_sc as plsc

SEQ_LEN = 512
CLS_ID = 101
SEP_ID = 102
PAD_ID = 0
B = 16
TOTAL = 4096
BUDGET = SEQ_LEN - 3
LANES = 16
WIN = 528
WCAP = TOTAL - WIN


def _pack_call(tokens1, cu1, tokens2, cu2, label):
    mesh = plsc.VectorSubcoreMesh(
        core_axis_name="c", subcore_axis_name="s", num_cores=1)
    out_sds = jax.ShapeDtypeStruct((B, SEQ_LEN), jnp.int32)
    lab_sds = jax.ShapeDtypeStruct((B,), jnp.float32)

    @functools.partial(
        pl.kernel,
        out_type=(out_sds, out_sds, out_sds, lab_sds),
        mesh=mesh,
        compiler_params=pltpu.CompilerParams(needs_layout_passes=False),
        scratch_types=[
            pltpu.VMEM((2 * WIN,), jnp.int32),
            pltpu.VMEM((32,), jnp.int32),
            pltpu.VMEM((32,), jnp.int32),
            pltpu.VMEM((SEQ_LEN,), jnp.int32),
            pltpu.VMEM((SEQ_LEN,), jnp.int32),
            pltpu.VMEM((SEQ_LEN,), jnp.int32),
            pltpu.VMEM((B,), jnp.float32),
            pltpu.SemaphoreType.DMA,
        ],
    )
    def body(tok1_hbm, cu1_hbm, tok2_hbm, cu2_hbm, lab_hbm,
             ids_hbm, mask_hbm, tid_hbm, lab_out_hbm,
             tok_v, cu1_v, cu2_v, ids_v, mask_v, tid_v, lab_v, sem):
        r = lax.axis_index("s")

        cp_a = pltpu.async_copy(cu1_hbm, cu1_v.at[pl.ds(0, B + 1)], sem)
        cp_b = pltpu.async_copy(cu2_hbm, cu2_v.at[pl.ds(0, B + 1)], sem)

        @pl.when(r == 0)
        def _label_in():
            pltpu.async_copy(lab_hbm, lab_v, sem).wait()

        cp_a.wait()
        cp_b.wait()

        cu1_vec = cu1_v[pl.ds(r, LANES)]
        cu2_vec = cu2_v[pl.ds(r, LANES)]
        c1 = cu1_vec[0]
        len1 = cu1_vec[1] - c1
        c2 = cu2_vec[0]
        len2 = cu2_vec[1] - c2
        t1 = jnp.minimum(len1, BUDGET - jnp.minimum(len2, BUDGET // 2))
        t2 = jnp.minimum(len2, BUDGET - t1)

        s1 = jnp.minimum(jnp.bitwise_and(c1, -8), WCAP)
        s2 = jnp.minimum(jnp.bitwise_and(c2, -8), WCAP)
        cp_1 = pltpu.async_copy(
            tok1_hbm.at[pl.ds(pl.multiple_of(s1, 8), WIN)],
            tok_v.at[pl.ds(0, WIN)], sem)
        cp_2 = pltpu.async_copy(
            tok2_hbm.at[pl.ds(pl.multiple_of(s2, 8), WIN)],
            tok_v.at[pl.ds(WIN, WIN)], sem)

        @pl.when(r == 0)
        def _label_out():
            pltpu.async_copy(lab_v, lab_out_hbm, sem).wait()

        cp_1.wait()
        cp_2.wait()

        a1 = c1 - 1 - s1
        a2 = c2 - s2 + WIN
        t1u = jnp.uint32(t1)
        t2u = jnp.uint32(t2)

        @plsc.parallel_loop(0, SEQ_LEN, LANES, unroll=4)
        def step(off):
            p = off + lax.iota(jnp.int32, LANES)
            d2 = p - (t1 + 2)
            in1 = (p - 1).astype(jnp.uint32) < t1u
            in2 = d2.astype(jnp.uint32) < t2u
            g1 = plsc.load_gather(tok_v, [jnp.clip(p + a1, 0, WIN - 1)])
            g2 = plsc.load_gather(tok_v, [jnp.clip(d2 + a2, WIN, 2 * WIN - 1)])
            sep = (d2 == -1) | (d2 == t2)
            ids_v[pl.ds(off, LANES)] = jnp.where(in1, g1,
                jnp.where(in2, g2,
                jnp.where(sep, SEP_ID, PAD_ID))).astype(jnp.int32)
            mask_v[pl.ds(off, LANES)] = (d2 <= t2).astype(jnp.int32)
            tid_v[pl.ds(off, LANES)] = (in2 | (d2 == t2)).astype(jnp.int32)

        head = ids_v[pl.ds(0, LANES)]
        ids_v[pl.ds(0, LANES)] = jnp.where(
            lax.iota(jnp.int32, LANES) == 0, CLS_ID, head)

        cp_o1 = pltpu.async_copy(ids_v, ids_hbm.at[r], sem)
        cp_o2 = pltpu.async_copy(mask_v, mask_hbm.at[r], sem)
        cp_o3 = pltpu.async_copy(tid_v, tid_hbm.at[r], sem)
        cp_o1.wait()
        cp_o2.wait()
        cp_o3.wait()

    return body(tokens1, cu1, tokens2, cu2, label)


def kernel(tokens1, cu_seqlens1, tokens2, cu_seqlens2, label):
    ids, mask, tids, lab = _pack_call(
        tokens1, cu_seqlens1, tokens2, cu_seqlens2, label)
    return (ids, mask, tids, lab)


# speedup vs baseline: 1.0050x; 1.0050x over previous
import functools

import jax
import jax.numpy as jnp
from jax import lax
from jax.experimental import pallas as pl
from jax.experimental.pallas import tpu as pltpu
from jax.experimental.pallas import tpu_sc as plsc

SEQ_LEN = 512
CLS_ID = 101
SEP_ID = 102
PAD_ID = 0
B = 16
TOTAL = 4096
BUDGET = SEQ_LEN - 3
LANES = 16
WIN = 528
WCAP = TOTAL - WIN


def _pack_call(tokens1, cu1, tokens2, cu2, label):
    mesh = plsc.VectorSubcoreMesh(
        core_axis_name="c", subcore_axis_name="s", num_cores=1)
    out_sds = jax.ShapeDtypeStruct((B, SEQ_LEN), jnp.int32)
    lab_sds = jax.ShapeDtypeStruct((B,), jnp.float32)

    @functools.partial(
        pl.kernel,
        out_type=(out_sds, out_sds, out_sds, lab_sds),
        mesh=mesh,
        compiler_params=pltpu.CompilerParams(needs_layout_passes=False),
        scratch_types=[
            pltpu.VMEM((2 * WIN,), jnp.int32),
            pltpu.VMEM((32,), jnp.int32),
            pltpu.VMEM((32,), jnp.int32),
            pltpu.VMEM((SEQ_LEN,), jnp.int32),
            pltpu.VMEM((SEQ_LEN,), jnp.int32),
            pltpu.VMEM((SEQ_LEN,), jnp.int32),
            pltpu.VMEM((B,), jnp.float32),
            pltpu.SemaphoreType.DMA,
        ],
    )
    def body(tok1_hbm, cu1_hbm, tok2_hbm, cu2_hbm, lab_hbm,
             ids_hbm, mask_hbm, tid_hbm, lab_out_hbm,
             tok_v, cu1_v, cu2_v, ids_v, mask_v, tid_v, lab_v, sem):
        r = lax.axis_index("s")

        cp_a = pltpu.async_copy(cu1_hbm, cu1_v.at[pl.ds(0, B + 1)], sem)
        cp_b = pltpu.async_copy(cu2_hbm, cu2_v.at[pl.ds(0, B + 1)], sem)

        @pl.when(r == 0)
        def _label_in():
            pltpu.async_copy(lab_hbm, lab_v, sem).wait()

        cp_a.wait()
        cp_b.wait()

        cu1_vec = cu1_v[pl.ds(r, LANES)]
        cu2_vec = cu2_v[pl.ds(r, LANES)]
        c1 = cu1_vec[0]
        len1 = cu1_vec[1] - c1
        c2 = cu2_vec[0]
        len2 = cu2_vec[1] - c2
        t1 = jnp.minimum(len1, BUDGET - jnp.minimum(len2, BUDGET // 2))
        t2 = jnp.minimum(len2, BUDGET - t1)

        s1 = jnp.minimum(jnp.bitwise_and(c1, -8), WCAP)
        s2 = jnp.minimum(jnp.bitwise_and(c2, -8), WCAP)
        cp_1 = pltpu.async_copy(
            tok1_hbm.at[pl.ds(pl.multiple_of(s1, 8), WIN)],
            tok_v.at[pl.ds(0, WIN)], sem)
        cp_2 = pltpu.async_copy(
            tok2_hbm.at[pl.ds(pl.multiple_of(s2, 8), WIN)],
            tok_v.at[pl.ds(WIN, WIN)], sem)

        @pl.when(r == 0)
        def _label_out():
            pltpu.async_copy(lab_v, lab_out_hbm, sem).wait()

        cp_1.wait()
        cp_2.wait()

        a1 = c1 - 1 - s1
        a2 = c2 - s2 + WIN
        t1u = jnp.uint32(t1)
        t2u = jnp.uint32(t2)

        @plsc.parallel_loop(0, SEQ_LEN, LANES, unroll=1)
        def step(off):
            p = off + lax.iota(jnp.int32, LANES)
            d2 = p - (t1 + 2)
            in1 = (p - 1).astype(jnp.uint32) < t1u
            in2 = d2.astype(jnp.uint32) < t2u
            g1 = plsc.load_gather(tok_v, [jnp.clip(p + a1, 0, WIN - 1)])
            g2 = plsc.load_gather(tok_v, [jnp.clip(d2 + a2, WIN, 2 * WIN - 1)])
            sep = (d2 == -1) | (d2 == t2)
            ids_v[pl.ds(off, LANES)] = jnp.where(in1, g1,
                jnp.where(in2, g2,
                jnp.where(sep, SEP_ID, PAD_ID))).astype(jnp.int32)
            mask_v[pl.ds(off, LANES)] = (d2 <= t2).astype(jnp.int32)
            tid_v[pl.ds(off, LANES)] = (in2 | (d2 == t2)).astype(jnp.int32)

        head = ids_v[pl.ds(0, LANES)]
        ids_v[pl.ds(0, LANES)] = jnp.where(
            lax.iota(jnp.int32, LANES) == 0, CLS_ID, head)

        cp_o1 = pltpu.async_copy(ids_v, ids_hbm.at[r], sem)
        cp_o2 = pltpu.async_copy(mask_v, mask_hbm.at[r], sem)
        cp_o3 = pltpu.async_copy(tid_v, tid_hbm.at[r], sem)
        cp_o1.wait()
        cp_o2.wait()
        cp_o3.wait()

    return body(tokens1, cu1, tokens2, cu2, label)


def kernel(tokens1, cu_seqlens1, tokens2, cu_seqlens2, label):
    ids, mask, tids, lab = _pack_call(
        tokens1, cu_seqlens1, tokens2, cu_seqlens2, label)
    return (ids, mask, tids, lab)
